# baseline (speedup 1.0000x reference)
import jax
import jax.numpy as jnp
from jax import lax
from jax.experimental import pallas as pl
from jax.experimental.pallas import tpu as pltpu

_B, _F, _N, _D = 8, 2, 1024, 16
_K = 20
_MM = jnp.bfloat16
_DN = (((1,), (1,)), ((), ()))

_NOISE = None


def _noise_const():
    global _NOISE
    if _NOISE is not None:
        return _NOISE
    import numpy as np

    def rotl(x, r):
        return ((x << np.uint32(r)) | (x >> np.uint32(32 - r))).astype(np.uint32)

    n = _B * _N * _N
    i = np.arange(n, dtype=np.uint64)
    x0 = (i >> np.uint64(32)).astype(np.uint32)
    x1 = (i & np.uint64(0xFFFFFFFF)).astype(np.uint32)
    k0, k1 = np.uint32(0), np.uint32(42)
    ks = [k0, k1, np.uint32(k0 ^ k1 ^ np.uint32(0x1BD11BDA))]
    rotations = [[13, 15, 26, 6], [17, 29, 16, 24]]
    x0 = (x0 + ks[0]).astype(np.uint32)
    x1 = (x1 + ks[1]).astype(np.uint32)
    for r in range(5):
        for rot in rotations[r % 2]:
            x0 = (x0 + x1).astype(np.uint32)
            x1 = rotl(x1, rot) ^ x0
        x0 = (x0 + ks[(r + 1) % 3]).astype(np.uint32)
        x1 = (x1 + ks[(r + 2) % 3] + np.uint32(r + 1)).astype(np.uint32)
    bits = x0 ^ x1
    f = ((bits >> np.uint32(9)) | np.uint32(0x3F800000)).view(np.float32)
    f = np.maximum(np.float32(0.0), f - np.float32(1.0))
    _NOISE = (f * np.float32(0.01)).reshape(_B, _N, _N)
    return _NOISE


def _body(x1_ref, x2_ref, noise_ref, out_ref, work_ref):
    def nodevec(xref):
        c = jnp.concatenate([xref[0, 0], xref[0, 1]], axis=1).astype(_MM)
        s = lax.dot_general(c, c, _DN, preferred_element_type=jnp.float32)
        return jnp.tanh(s * 0.5)

    nv1 = nodevec(x1_ref).astype(_MM)
    nv2 = nodevec(x2_ref).astype(_MM)
    adj = (lax.dot_general(nv1, nv2, _DN, preferred_element_type=jnp.float32)
           - lax.dot_general(nv2, nv1, _DN, preferred_element_type=jnp.float32))
    out_ref[0] = adj
    work_ref[...] = adj + noise_ref[0]

    def comb(a1, a2, b1, b2):
        return (jnp.maximum(a1, b1),
                jnp.maximum(jnp.minimum(a1, b1), jnp.maximum(a2, b2)))

    def fold2(m1, m2):
        while m1.shape[0] > 1:
            n = m1.shape[0] // 2
            m1, m2 = comb(m1[:n], m2[:n], m1[n:], m2[n:])
        return m1, m2

    def step(_, t):
        p1, p2 = [], []
        for c in range(16):
            w = work_ref[c * 64:(c + 1) * 64, :].reshape(8, 8, _N)
            w = jnp.where(w >= t, -jnp.inf, w)
            a, b = w[:4], w[4:]
            m1, m2 = fold2(jnp.maximum(a, b), jnp.minimum(a, b))
            p1.append(m1)
            p2.append(m2)
        while len(p1) > 1:
            p1, p2 = map(list, zip(*[
                comb(p1[i], p2[i], p1[i + 1], p2[i + 1])
                for i in range(0, len(p1), 2)]))
        m1, m2 = fold2(p1[0].reshape(8, _N), p2[0].reshape(8, _N))
        return m2.reshape(1, 1, _N)

    t20 = lax.fori_loop(0, _K // 2, step,
                        jnp.full((1, 1, _N), jnp.inf, jnp.float32))
    adj2 = out_ref[0]
    out_ref[0] = jnp.where(work_ref[...] >= t20.reshape(1, _N), adj2, 0.0)


def _run(x1, x2, noise):
    return pl.pallas_call(
        _body,
        grid=(_B,),
        in_specs=[
            pl.BlockSpec((1, _F, _N, _D), lambda b: (b, 0, 0, 0)),
            pl.BlockSpec((1, _F, _N, _D), lambda b: (b, 0, 0, 0)),
            pl.BlockSpec((1, _N, _N), lambda b: (b, 0, 0)),
        ],
        out_specs=pl.BlockSpec((1, _N, _N), lambda b: (b, 0, 0)),
        out_shape=jax.ShapeDtypeStruct((_B, _N, _N), jnp.float32),
        scratch_shapes=[pltpu.VMEM((_N, _N), jnp.float32)],
    )(x1, x2, noise)


def kernel(idx, time_in_day_feat, day_in_week_feat, emb1_table, emb2_table):
    return _run(time_in_day_feat, day_in_week_feat, _noise_const())


# speedup vs baseline: 1.0674x; 1.0674x over previous
import jax
import jax.numpy as jnp
from jax import lax
from jax.experimental import pallas as pl
from jax.experimental.pallas import tpu as pltpu

_B, _F, _N, _D = 8, 2, 1024, 16
_K = 20
_MM = jnp.bfloat16
_DN = (((1,), (1,)), ((), ()))

_NOISE = None


def _noise_const():
    global _NOISE
    if _NOISE is not None:
        return _NOISE
    import numpy as np

    def rotl(x, r):
        return ((x << np.uint32(r)) | (x >> np.uint32(32 - r))).astype(np.uint32)

    n = _B * _N * _N
    i = np.arange(n, dtype=np.uint64)
    x0 = (i >> np.uint64(32)).astype(np.uint32)
    x1 = (i & np.uint64(0xFFFFFFFF)).astype(np.uint32)
    k0, k1 = np.uint32(0), np.uint32(42)
    ks = [k0, k1, np.uint32(k0 ^ k1 ^ np.uint32(0x1BD11BDA))]
    rotations = [[13, 15, 26, 6], [17, 29, 16, 24]]
    x0 = (x0 + ks[0]).astype(np.uint32)
    x1 = (x1 + ks[1]).astype(np.uint32)
    for r in range(5):
        for rot in rotations[r % 2]:
            x0 = (x0 + x1).astype(np.uint32)
            x1 = rotl(x1, rot) ^ x0
        x0 = (x0 + ks[(r + 1) % 3]).astype(np.uint32)
        x1 = (x1 + ks[(r + 2) % 3] + np.uint32(r + 1)).astype(np.uint32)
    bits = x0 ^ x1
    f = ((bits >> np.uint32(9)) | np.uint32(0x3F800000)).view(np.float32)
    f = np.maximum(np.float32(0.0), f - np.float32(1.0))
    _NOISE = (f * np.float32(0.01)).reshape(_B, _N, _N)
    return _NOISE


def _body(x1_ref, x2_ref, noise_ref, out_ref, work_ref):
    def nodevec(xref):
        c = jnp.concatenate([xref[0, 0], xref[0, 1]], axis=1).astype(_MM)
        s = lax.dot_general(c, c, _DN, preferred_element_type=jnp.float32)
        return jnp.tanh(s * 0.5)

    nv1 = nodevec(x1_ref).astype(_MM)
    nv2 = nodevec(x2_ref).astype(_MM)
    adj = (lax.dot_general(nv1, nv2, _DN, preferred_element_type=jnp.float32)
           - lax.dot_general(nv2, nv1, _DN, preferred_element_type=jnp.float32))
    work_ref[...] = adj + noise_ref[0]

    def step(_, t):
        parts = []
        for c in range(16):
            w = work_ref[c * 64:(c + 1) * 64, :].reshape(8, 8, _N)
            w = jnp.where(w >= t, -jnp.inf, w)
            parts.append(jnp.max(w, axis=0))
        while len(parts) > 1:
            parts = [jnp.maximum(parts[i], parts[i + 1])
                     for i in range(0, len(parts), 2)]
        return jnp.max(parts[0], axis=0, keepdims=True).reshape(1, 1, _N)

    t20 = lax.fori_loop(0, _K, step,
                        jnp.full((1, 1, _N), jnp.inf, jnp.float32))
    w = work_ref[...]
    out_ref[0] = jnp.where(w >= t20.reshape(1, _N), w - noise_ref[0], 0.0)


def _run(x1, x2, noise):
    return pl.pallas_call(
        _body,
        grid=(_B,),
        in_specs=[
            pl.BlockSpec((1, _F, _N, _D), lambda b: (b, 0, 0, 0)),
            pl.BlockSpec((1, _F, _N, _D), lambda b: (b, 0, 0, 0)),
            pl.BlockSpec((1, _N, _N), lambda b: (b, 0, 0)),
        ],
        out_specs=pl.BlockSpec((1, _N, _N), lambda b: (b, 0, 0)),
        out_shape=jax.ShapeDtypeStruct((_B, _N, _N), jnp.float32),
        scratch_shapes=[pltpu.VMEM((_N, _N), jnp.float32)],
    )(x1, x2, noise)


def kernel(idx, time_in_day_feat, day_in_week_feat, emb1_table, emb2_table):
    return _run(time_in_day_feat, day_in_week_feat, _noise_const())
